# baseline (speedup 1.0000x reference)
import functools

import numpy as np
import jax
import jax.numpy as jnp
from jax import lax
from jax.experimental import pallas as pl
from jax.experimental.pallas import tpu as pltpu
from jax.experimental.pallas import tpu_sc as plsc

N = 10000
E = 160000
D = 256
H = 32
G = 64

NC = 2
NS = 16
NW = NC * NS
NP = 10112
NP4 = NP // 4
N4 = N // 4
EP = 163840
EPW = EP // NW
CH = 128
NCH = EPW // CH
R4 = NP4 // NS

_MESH = plsc.VectorSubcoreMesh(core_axis_name="c", subcore_axis_name="s")
_SC_PARAMS = pltpu.CompilerParams(use_tc_tiling_on_sc=False)
RPT = NP // NS


@functools.partial(
    pl.kernel,
    out_type=jax.ShapeDtypeStruct((NC, NP, H), jnp.float32),
    mesh=_MESH,
    compiler_params=_SC_PARAMS,
    scratch_types=[
        pltpu.VMEM((NCH, CH), jnp.int32),
        pltpu.VMEM((CH, H), jnp.float32),
        pltpu.VMEM_SHARED((NP, H), jnp.float32),
        pltpu.SemaphoreType.DMA,
    ],
)
def _deg_kernel(edge_hbm, ones_hbm, zeros_hbm, out_hbm, idx_v, ones_v, acc, sem):
    c = lax.axis_index("c")
    s = lax.axis_index("s")
    wid = c * NS + s
    pltpu.async_copy(zeros_hbm.at[pl.ds(s * RPT, RPT)], acc.at[pl.ds(s * RPT, RPT)], sem)
    pltpu.sync_copy(ones_hbm, ones_v)
    pltpu.sync_copy(edge_hbm.at[1].at[wid], idx_v)
    pltpu.make_async_copy(zeros_hbm.at[pl.ds(s * RPT, RPT)],
                          acc.at[pl.ds(s * RPT, RPT)], sem).wait()
    plsc.subcore_barrier()

    def fire(g, carry):
        pltpu.async_copy(ones_v, acc.at[idx_v.at[g]], sem, add=True)
        return carry

    lax.fori_loop(0, NCH, fire, 0)

    def drain(g, carry):
        pltpu.make_async_copy(ones_v, acc.at[idx_v.at[0]], sem).wait()
        return carry

    lax.fori_loop(0, NCH, drain, 0)
    plsc.subcore_barrier()
    pltpu.sync_copy(acc.at[pl.ds(s * RPT, RPT)], out_hbm.at[c].at[pl.ds(s * RPT, RPT)])


NB = 4
GAH = 2

@functools.partial(
    pl.kernel,
    out_type=jax.ShapeDtypeStruct((NC, NP, H), jnp.float32),
    mesh=_MESH,
    compiler_params=_SC_PARAMS,
    scratch_types=[
        pltpu.VMEM((NCH, CH), jnp.int32),
        pltpu.VMEM((NCH, CH), jnp.int32),
        *[pltpu.VMEM((CH, H), jnp.float32) for _ in range(NB)],
        pltpu.VMEM_SHARED((NP, H), jnp.float32),
        *[pltpu.SemaphoreType.DMA for _ in range(2 * NB)],
    ],
)
def _agg_kernel(u_hbm, zeros_hbm, edge_hbm, out_hbm,
                sidx_v, didx_v, *rest):
    rows = rest[:NB]
    acc = rest[NB]
    gsem = rest[NB + 1:2 * NB + 1]
    ssem = rest[2 * NB + 1:]
    c = lax.axis_index("c")
    s = lax.axis_index("s")
    wid = c * NS + s

    isem = gsem[0]

    @pl.when(c == 0)
    def _():
        pltpu.async_copy(u_hbm.at[pl.ds(s * RPT, RPT)], acc.at[pl.ds(s * RPT, RPT)], isem)

    @pl.when(c != 0)
    def _():
        pltpu.async_copy(zeros_hbm.at[pl.ds(s * RPT, RPT)], acc.at[pl.ds(s * RPT, RPT)], isem)

    pltpu.sync_copy(edge_hbm.at[0].at[wid], sidx_v)
    pltpu.sync_copy(edge_hbm.at[1].at[wid], didx_v)
    pltpu.make_async_copy(u_hbm.at[pl.ds(s * RPT, RPT)],
                          acc.at[pl.ds(s * RPT, RPT)], isem).wait()
    plsc.subcore_barrier()

    def gfire(g, k):
        pltpu.async_copy(u_hbm.at[sidx_v.at[g]], rows[k], gsem[k])

    def gwait(g, k):
        pltpu.make_async_copy(u_hbm.at[sidx_v.at[g]], rows[k], gsem[k]).wait()

    def sfire(g, k):
        pltpu.async_copy(rows[k], acc.at[didx_v.at[g]], ssem[k], add=True)

    def swait(k):
        pltpu.make_async_copy(rows[k], acc.at[didx_v.at[0]], ssem[k]).wait()

    for k in range(NB):
        gfire(k, k)
        if k >= GAH:
            kb = k - GAH
            gwait(kb, kb)
            sfire(kb, kb)

    def body(p, carry):
        for k in range(NB):
            g = NB * p + k
            swait(k)
            gfire(g, k)
            kb = (k + GAH) % NB
            gwait(g - GAH, kb)
            sfire(g - GAH, kb)
        return carry

    lax.fori_loop(1, NCH // NB, body, 0)

    for k in range(GAH, NB):
        g = NCH - NB + k
        gwait(g, k)
        sfire(g, k)
    for k in range(NB):
        swait(k)

    plsc.subcore_barrier()
    pltpu.sync_copy(acc.at[pl.ds(s * RPT, RPT)], out_hbm.at[c].at[pl.ds(s * RPT, RPT)])


def _m1_body(x_ref, w1_ref, xw_ref):
    xb = x_ref[...].astype(jnp.bfloat16)
    wb = w1_ref[...].astype(jnp.bfloat16)
    xw = jnp.dot(xb, wb, preferred_element_type=jnp.float32)
    xw_ref[0:N, :] = xw
    xw_ref[N:NP, :] = jnp.zeros((NP - N, H), jnp.float32)


def _m1(x, w1):
    return pl.pallas_call(
        _m1_body,
        out_shape=jax.ShapeDtypeStruct((NP, H), jnp.float32),
    )(x, w1)


def _m1b_body(xw_ref, deg_ref, u1_ref):
    dinv = lax.rsqrt(1.0 + deg_ref[0] + deg_ref[1])
    u1_ref[...] = xw_ref[...] * dinv


def _m1b(xw_lin, deg):
    return pl.pallas_call(
        _m1b_body,
        out_shape=jax.ShapeDtypeStruct((NP4, 128), jnp.float32),
    )(xw_lin, deg)


def _m2_body(p_ref, deg_ref, b1_ref, w2blk_ref, u2_ref):
    dinv = lax.rsqrt(1.0 + deg_ref[0] + deg_ref[1])
    agg = p_ref[0] + p_ref[1]
    h1 = jnp.maximum(agg * dinv + b1_ref[...], 0.0)
    u2 = jnp.dot(h1, w2blk_ref[...], preferred_element_type=jnp.float32) * dinv
    rid = lax.broadcasted_iota(jnp.int32, (NP4, 128), 0)
    u2_ref[...] = jnp.where(rid < N4, u2, 0.0)


def _m2(p, deg, b1_lin, w2blk):
    return pl.pallas_call(
        _m2_body,
        out_shape=jax.ShapeDtypeStruct((NP4, 128), jnp.float32),
    )(p, deg, b1_lin, w2blk)


def _m3_body(q_ref, deg_ref, b2_ref, batch4_ref,
             wfc_ref, bfc_ref, wreg_ref, breg_ref, wcls_ref, bcls_ref,
             reg_ref, cls_ref):
    dinv = lax.rsqrt(1.0 + deg_ref[0] + deg_ref[1])
    agg = q_ref[0] + q_ref[1]
    h2 = jnp.maximum(agg * dinv + b2_ref[...], 0.0)
    gids = lax.broadcasted_iota(jnp.int32, (G, NP4), 0)
    sums = jnp.zeros((G, H), jnp.float32)
    cnt = jnp.zeros((G, 1), jnp.float32)
    for j in range(4):
        oh = (batch4_ref[j:j + 1, :] == gids).astype(jnp.float32)
        sj = jnp.dot(oh, h2, preferred_element_type=jnp.float32)
        sums = sums + sj[:, 32 * j:32 * (j + 1)]
        cnt = cnt + jnp.sum(oh, axis=1, keepdims=True)
    pooled = sums / jnp.maximum(cnt, 1.0)
    s = jnp.maximum(
        jnp.dot(pooled, wfc_ref[...], preferred_element_type=jnp.float32)
        + bfc_ref[...], 0.0)
    reg_ref[...] = jnp.dot(s, wreg_ref[...],
                           preferred_element_type=jnp.float32) + breg_ref[...]
    cls_ref[...] = jnp.dot(s, wcls_ref[...],
                           preferred_element_type=jnp.float32) + bcls_ref[...]


def _m3(q, deg, b2_lin, batch4, wfc, bfc, wreg, breg, wcls, bcls):
    return pl.pallas_call(
        _m3_body,
        out_shape=(
            jax.ShapeDtypeStruct((G, 2), jnp.float32),
            jax.ShapeDtypeStruct((G, 2), jnp.float32),
        ),
    )(q, deg, b2_lin, batch4, wfc, bfc, wreg, breg, wcls, bcls)


def kernel(x, edge_index, batch, W1, b1, W2, b2, Wfc, bfc, Wreg, breg, Wcls, bcls):
    pr = np.arange(EP - E, dtype=np.int32) % (NP - N)
    padc = jnp.asarray(np.stack([N + pr, N + (NP - N - 1) - pr]))
    edgep = jnp.concatenate([edge_index, padc], axis=1).reshape(2, NW, NCH, CH)
    zeros_h = jnp.zeros((NP, H), jnp.float32)
    ones_h = jnp.ones((CH, H), jnp.float32)
    w2blk = jnp.kron(jnp.eye(4, dtype=jnp.float32), W2)
    b1_lin = jnp.tile(b1, 4).reshape(1, 128)
    b2_lin = jnp.tile(b2, 4).reshape(1, 128)
    batch4 = jnp.concatenate(
        [batch.reshape(N4, 4).T,
         jnp.full((4, NP4 - N4), -1, jnp.int32)], axis=1)

    deg = _deg_kernel(edgep, ones_h, zeros_h).reshape(NC, NP4, 128)
    xw_lin = _m1(x, W1).reshape(NP4, 128)
    u1 = _m1b(xw_lin, deg).reshape(NP, H)
    p1 = _agg_kernel(u1, zeros_h, edgep).reshape(NC, NP4, 128)
    u2 = _m2(p1, deg, b1_lin, w2blk).reshape(NP, H)
    p2 = _agg_kernel(u2, zeros_h, edgep).reshape(NC, NP4, 128)
    reg, cls = _m3(p2, deg, b2_lin, batch4,
                   Wfc, bfc.reshape(1, H), Wreg, breg.reshape(1, 2),
                   Wcls, bcls.reshape(1, 2))
    return (reg, cls)


# speedup vs baseline: 1.0290x; 1.0290x over previous
import functools

import numpy as np
import jax
import jax.numpy as jnp
from jax import lax
from jax.experimental import pallas as pl
from jax.experimental.pallas import tpu as pltpu
from jax.experimental.pallas import tpu_sc as plsc

N = 10000
E = 160000
D = 256
H = 32
G = 64

NC = 2
NS = 16
NW = NC * NS
NP = 10112
NP4 = NP // 4
N4 = N // 4
EP = 163840
EPW = EP // NW
CH = 128
NCH = EPW // CH
R4 = NP4 // NS

_MESH = plsc.VectorSubcoreMesh(core_axis_name="c", subcore_axis_name="s")
_SC_PARAMS = pltpu.CompilerParams(use_tc_tiling_on_sc=False)
RPT = NP // NS


@functools.partial(
    pl.kernel,
    out_type=jax.ShapeDtypeStruct((NC, NP, H), jnp.float32),
    mesh=_MESH,
    compiler_params=_SC_PARAMS,
    scratch_types=[
        pltpu.VMEM((NCH, CH), jnp.int32),
        pltpu.VMEM((CH, H), jnp.float32),
        pltpu.VMEM_SHARED((NP, H), jnp.float32),
        pltpu.SemaphoreType.DMA,
    ],
)
def _deg_kernel(edge_hbm, ones_hbm, zeros_hbm, out_hbm, idx_v, ones_v, acc, sem):
    c = lax.axis_index("c")
    s = lax.axis_index("s")
    wid = c * NS + s
    pltpu.async_copy(zeros_hbm.at[pl.ds(s * RPT, RPT)], acc.at[pl.ds(s * RPT, RPT)], sem)
    pltpu.sync_copy(ones_hbm, ones_v)
    pltpu.sync_copy(edge_hbm.at[1].at[wid], idx_v)
    pltpu.make_async_copy(zeros_hbm.at[pl.ds(s * RPT, RPT)],
                          acc.at[pl.ds(s * RPT, RPT)], sem).wait()
    plsc.subcore_barrier()

    def fire(g, carry):
        pltpu.async_copy(ones_v, acc.at[idx_v.at[g]], sem, add=True)
        return carry

    lax.fori_loop(0, NCH, fire, 0)

    def drain(g, carry):
        pltpu.make_async_copy(ones_v, acc.at[idx_v.at[0]], sem).wait()
        return carry

    lax.fori_loop(0, NCH, drain, 0)
    plsc.subcore_barrier()
    pltpu.sync_copy(acc.at[pl.ds(s * RPT, RPT)], out_hbm.at[c].at[pl.ds(s * RPT, RPT)])


NB = 8
GAH = 4

@functools.partial(
    pl.kernel,
    out_type=jax.ShapeDtypeStruct((NC, NP, H), jnp.float32),
    mesh=_MESH,
    compiler_params=_SC_PARAMS,
    scratch_types=[
        pltpu.VMEM((NCH, CH), jnp.int32),
        pltpu.VMEM((NCH, CH), jnp.int32),
        *[pltpu.VMEM((CH, H), jnp.float32) for _ in range(NB)],
        pltpu.VMEM_SHARED((NP, H), jnp.float32),
        *[pltpu.SemaphoreType.DMA for _ in range(2 * NB)],
    ],
)
def _agg_kernel(u_hbm, zeros_hbm, edge_hbm, out_hbm,
                sidx_v, didx_v, *rest):
    rows = rest[:NB]
    acc = rest[NB]
    gsem = rest[NB + 1:2 * NB + 1]
    ssem = rest[2 * NB + 1:]
    c = lax.axis_index("c")
    s = lax.axis_index("s")
    wid = c * NS + s

    isem = gsem[0]

    @pl.when(c == 0)
    def _():
        pltpu.async_copy(u_hbm.at[pl.ds(s * RPT, RPT)], acc.at[pl.ds(s * RPT, RPT)], isem)

    @pl.when(c != 0)
    def _():
        pltpu.async_copy(zeros_hbm.at[pl.ds(s * RPT, RPT)], acc.at[pl.ds(s * RPT, RPT)], isem)

    pltpu.sync_copy(edge_hbm.at[0].at[wid], sidx_v)
    pltpu.sync_copy(edge_hbm.at[1].at[wid], didx_v)
    pltpu.make_async_copy(u_hbm.at[pl.ds(s * RPT, RPT)],
                          acc.at[pl.ds(s * RPT, RPT)], isem).wait()
    plsc.subcore_barrier()

    def gfire(g, k):
        pltpu.async_copy(u_hbm.at[sidx_v.at[g]], rows[k], gsem[k])

    def gwait(g, k):
        pltpu.make_async_copy(u_hbm.at[sidx_v.at[g]], rows[k], gsem[k]).wait()

    def sfire(g, k):
        pltpu.async_copy(rows[k], acc.at[didx_v.at[g]], ssem[k], add=True)

    def swait(k):
        pltpu.make_async_copy(rows[k], acc.at[didx_v.at[0]], ssem[k]).wait()

    for k in range(NB):
        gfire(k, k)
        if k >= GAH:
            kb = k - GAH
            gwait(kb, kb)
            sfire(kb, kb)

    def body(p, carry):
        for k in range(NB):
            g = NB * p + k
            swait(k)
            gfire(g, k)
            kb = (k + GAH) % NB
            gwait(g - GAH, kb)
            sfire(g - GAH, kb)
        return carry

    lax.fori_loop(1, NCH // NB, body, 0)

    for k in range(GAH, NB):
        g = NCH - NB + k
        gwait(g, k)
        sfire(g, k)
    for k in range(NB):
        swait(k)

    plsc.subcore_barrier()
    pltpu.sync_copy(acc.at[pl.ds(s * RPT, RPT)], out_hbm.at[c].at[pl.ds(s * RPT, RPT)])


def _m1_body(x_ref, w1_ref, xw_ref):
    xb = x_ref[...].astype(jnp.bfloat16)
    wb = w1_ref[...].astype(jnp.bfloat16)
    xw = jnp.dot(xb, wb, preferred_element_type=jnp.float32)
    xw_ref[0:N, :] = xw
    xw_ref[N:NP, :] = jnp.zeros((NP - N, H), jnp.float32)


def _m1(x, w1):
    return pl.pallas_call(
        _m1_body,
        out_shape=jax.ShapeDtypeStruct((NP, H), jnp.float32),
    )(x, w1)


def _m1b_body(xw_ref, deg_ref, u1_ref):
    dinv = lax.rsqrt(1.0 + deg_ref[0] + deg_ref[1])
    u1_ref[...] = xw_ref[...] * dinv


def _m1b(xw_lin, deg):
    return pl.pallas_call(
        _m1b_body,
        out_shape=jax.ShapeDtypeStruct((NP4, 128), jnp.float32),
    )(xw_lin, deg)


def _m2_body(p_ref, deg_ref, b1_ref, w2blk_ref, u2_ref):
    dinv = lax.rsqrt(1.0 + deg_ref[0] + deg_ref[1])
    agg = p_ref[0] + p_ref[1]
    h1 = jnp.maximum(agg * dinv + b1_ref[...], 0.0)
    u2 = jnp.dot(h1, w2blk_ref[...], preferred_element_type=jnp.float32) * dinv
    rid = lax.broadcasted_iota(jnp.int32, (NP4, 128), 0)
    u2_ref[...] = jnp.where(rid < N4, u2, 0.0)


def _m2(p, deg, b1_lin, w2blk):
    return pl.pallas_call(
        _m2_body,
        out_shape=jax.ShapeDtypeStruct((NP4, 128), jnp.float32),
    )(p, deg, b1_lin, w2blk)


def _m3_body(q_ref, deg_ref, b2_ref, batch4_ref,
             wfc_ref, bfc_ref, wreg_ref, breg_ref, wcls_ref, bcls_ref,
             reg_ref, cls_ref):
    dinv = lax.rsqrt(1.0 + deg_ref[0] + deg_ref[1])
    agg = q_ref[0] + q_ref[1]
    h2 = jnp.maximum(agg * dinv + b2_ref[...], 0.0)
    gids = lax.broadcasted_iota(jnp.int32, (G, NP4), 0)
    sums = jnp.zeros((G, H), jnp.float32)
    cnt = jnp.zeros((G, 1), jnp.float32)
    for j in range(4):
        oh = (batch4_ref[j:j + 1, :] == gids).astype(jnp.float32)
        sj = jnp.dot(oh, h2, preferred_element_type=jnp.float32)
        sums = sums + sj[:, 32 * j:32 * (j + 1)]
        cnt = cnt + jnp.sum(oh, axis=1, keepdims=True)
    pooled = sums / jnp.maximum(cnt, 1.0)
    s = jnp.maximum(
        jnp.dot(pooled, wfc_ref[...], preferred_element_type=jnp.float32)
        + bfc_ref[...], 0.0)
    reg_ref[...] = jnp.dot(s, wreg_ref[...],
                           preferred_element_type=jnp.float32) + breg_ref[...]
    cls_ref[...] = jnp.dot(s, wcls_ref[...],
                           preferred_element_type=jnp.float32) + bcls_ref[...]


def _m3(q, deg, b2_lin, batch4, wfc, bfc, wreg, breg, wcls, bcls):
    return pl.pallas_call(
        _m3_body,
        out_shape=(
            jax.ShapeDtypeStruct((G, 2), jnp.float32),
            jax.ShapeDtypeStruct((G, 2), jnp.float32),
        ),
    )(q, deg, b2_lin, batch4, wfc, bfc, wreg, breg, wcls, bcls)


def kernel(x, edge_index, batch, W1, b1, W2, b2, Wfc, bfc, Wreg, breg, Wcls, bcls):
    pr = np.arange(EP - E, dtype=np.int32) % (NP - N)
    padc = jnp.asarray(np.stack([N + pr, N + (NP - N - 1) - pr]))
    edgep = jnp.concatenate([edge_index, padc], axis=1).reshape(2, NW, NCH, CH)
    zeros_h = jnp.zeros((NP, H), jnp.float32)
    ones_h = jnp.ones((CH, H), jnp.float32)
    w2blk = jnp.kron(jnp.eye(4, dtype=jnp.float32), W2)
    b1_lin = jnp.tile(b1, 4).reshape(1, 128)
    b2_lin = jnp.tile(b2, 4).reshape(1, 128)
    batch4 = jnp.concatenate(
        [batch.reshape(N4, 4).T,
         jnp.full((4, NP4 - N4), -1, jnp.int32)], axis=1)

    deg = _deg_kernel(edgep, ones_h, zeros_h).reshape(NC, NP4, 128)
    xw_lin = _m1(x, W1).reshape(NP4, 128)
    u1 = _m1b(xw_lin, deg).reshape(NP, H)
    p1 = _agg_kernel(u1, zeros_h, edgep).reshape(NC, NP4, 128)
    u2 = _m2(p1, deg, b1_lin, w2blk).reshape(NP, H)
    p2 = _agg_kernel(u2, zeros_h, edgep).reshape(NC, NP4, 128)
    reg, cls = _m3(p2, deg, b2_lin, batch4,
                   Wfc, bfc.reshape(1, H), Wreg, breg.reshape(1, 2),
                   Wcls, bcls.reshape(1, 2))
    return (reg, cls)
